# baseline (speedup 1.0000x reference)
import functools

import jax
import jax.numpy as jnp
from jax import lax
from jax.experimental import pallas as pl
from jax.experimental.pallas import tpu as pltpu
from jax.experimental.pallas import tpu_sc as plsc

_VOCAB = 100000
_VOCABP = 100352
_HID = 64
_NC = 10
_NCP = 16
_B = 4096
_L = 200
_CHUNK = 100

_NW = 32
_SPW = _B // _NW
_CPW = 2 * _SPW
_NBUF = 8

_PACK = _VOCABP // 8
_PGRID = 8
_LBLK = _VOCABP // _PGRID
_PBLK = _LBLK // 8


def _project_body(tabt_ref, w1_ref, w2p_ref, p_ref):
    mt = lax.dot_general(
        w1_ref[...], w2p_ref[...], (((0,), (1,)), ((), ())),
        preferred_element_type=jnp.float32,
    )
    d = lax.dot_general(
        tabt_ref[...], mt, (((0,), (0,)), ((), ())),
        preferred_element_type=jnp.float32,
    )
    d3 = d.reshape(_PBLK, 8, _NCP)
    p = jnp.concatenate([d3[:, j, :] for j in range(8)], axis=1)
    rows = lax.broadcasted_iota(jnp.int32, (_PBLK, 128), 0)
    lanes = lax.broadcasted_iota(jnp.int32, (_PBLK, 128), 1)
    is_row0 = (rows == 0) & (lanes < _NCP) & (pl.program_id(0) == 0)
    p_ref[...] = jnp.where(is_row0, 0.0, p)


def _project(tabt, w1, w2p):
    return pl.pallas_call(
        _project_body,
        grid=(_PGRID,),
        in_specs=[
            pl.BlockSpec((_HID, _LBLK), lambda i: (0, i)),
            pl.BlockSpec((_HID, _HID), lambda i: (0, 0)),
            pl.BlockSpec((_NCP, _HID), lambda i: (0, 0)),
        ],
        out_specs=pl.BlockSpec((_PBLK, 128), lambda i: (i, 0)),
        out_shape=jax.ShapeDtypeStruct((_PACK, 128), jnp.float32),
    )(tabt, w1, w2p)


def _sc_pool_body(p_hbm, textt_hbm, out_hbm, idx_v, buf_v, acc_v, sem):
    c = lax.axis_index("c")
    s = lax.axis_index("s")
    w = s * 2 + c

    pltpu.sync_copy(textt_hbm.at[:, pl.ds(w * _SPW, _SPW)], idx_v)

    zero = jnp.zeros((_NCP,), jnp.float32)
    for r in range(_SPW):
        acc_v[r] = zero

    def issue(l, t):
        return pltpu.async_copy(p_hbm.at[idx_v.at[l]], buf_v.at[t], sem.at[t])

    def wait(l, t):
        pltpu.make_async_copy(p_hbm.at[idx_v.at[l]], buf_v.at[t], sem.at[t]).wait()

    for t in range(_NBUF - 1):
        issue(jnp.int32(t), t)

    def body(i, carry):
        l0 = i * _NBUF
        for t in range(_NBUF):
            l = l0 + t
            ln = l + (_NBUF - 1)

            @pl.when(ln < _L)
            def _():
                issue(ln, (t + _NBUF - 1) % _NBUF)

            wait(l, t)
            for r in range(_SPW):
                plsc.addupdate(acc_v.at[r], buf_v[t, r])
        return carry

    lax.fori_loop(0, _L // _NBUF, body, 0)
    pltpu.sync_copy(acc_v, out_hbm.at[pl.ds(w * _SPW, _SPW)])


@functools.partial(
    pl.kernel,
    mesh=plsc.VectorSubcoreMesh(core_axis_name="c", subcore_axis_name="s"),
    out_type=jax.ShapeDtypeStruct((_B, _NCP), jnp.float32),
    compiler_params=pltpu.CompilerParams(use_tc_tiling_on_sc=False),
    scratch_types=[
        pltpu.VMEM((_L, _SPW), jnp.int32),
        pltpu.VMEM((_NBUF, _SPW, _NCP), jnp.float32),
        pltpu.VMEM((_SPW, _NCP), jnp.float32),
        pltpu.SemaphoreType.DMA((_NBUF,)),
    ],
)
def _sc_pool(p_hbm, textt_hbm, out_hbm, idx_v, buf_v, acc_v, sem):
    _sc_pool_body(p_hbm, textt_hbm, out_hbm, idx_v, buf_v, acc_v, sem)


def _finalize_body(zsum_ref, b1_ref, w2p_ref, b2p_ref, out_ref):
    bias = (
        jnp.dot(b1_ref[...], w2p_ref[...].T, preferred_element_type=jnp.float32)
        + b2p_ref[...]
    )
    z = zsum_ref[...] * (1.0 / _L) + bias
    cols = lax.broadcasted_iota(jnp.int32, (_B, _NCP), 1)
    z = jnp.where(cols < _NC, z, -1e30)
    m = jnp.max(z, axis=1, keepdims=True)
    e = jnp.exp(z - m)
    lse = jnp.log(jnp.sum(e, axis=1, keepdims=True))
    out = z - m - lse
    out_ref[...] = out[:, :_NC]


def _finalize(zsum, b1r, w2p, b2p):
    return pl.pallas_call(
        _finalize_body,
        out_shape=jax.ShapeDtypeStruct((_B, _NC), jnp.float32),
    )(zsum, b1r, w2p, b2p)


def kernel(text, text_lengths, table, W1, b1, W2, b2):
    del text_lengths
    w2p = jnp.zeros((_NCP, _HID), jnp.float32).at[:_NC].set(W2)
    b2p = jnp.zeros((1, _NCP), jnp.float32).at[0, :_NC].set(b2)
    p_packed = _project(table.T, W1, w2p)
    p_lin = p_packed.reshape(_VOCABP, _NCP)
    zsum = _sc_pool(p_lin, text.T)
    return _finalize(zsum, b1.reshape(1, _HID), w2p, b2p)


# speedup vs baseline: 1.7412x; 1.7412x over previous
import functools

import jax
import jax.numpy as jnp
from jax import lax
from jax.experimental import pallas as pl
from jax.experimental.pallas import tpu as pltpu
from jax.experimental.pallas import tpu_sc as plsc

_VOCAB = 100000
_VOCABP = 100352
_HID = 64
_NC = 10
_NCP = 16
_B = 4096
_L = 200
_CHUNK = 100

_NW = 32
_SPW = _B // _NW
_CPW = 2 * _SPW
_NBUF = 8

_PACK = _VOCABP // 8
_PGRID = 8
_LBLK = _VOCABP // _PGRID
_PBLK = _LBLK // 8


def _project_body(tabt_ref, w1_ref, w2p_ref, p_ref):
    mt = lax.dot_general(
        w1_ref[...], w2p_ref[...], (((0,), (1,)), ((), ())),
        preferred_element_type=jnp.float32,
    )
    dt = lax.dot_general(
        mt, tabt_ref[...], (((0,), (0,)), ((), ())),
        preferred_element_type=jnp.float32,
    )
    lanes = lax.broadcasted_iota(jnp.int32, (_NCP, _LBLK), 1)
    p_ref[...] = jnp.where((lanes == 0) & (pl.program_id(0) == 0), 0.0, dt)


def _project(tabt, w1, w2p):
    return pl.pallas_call(
        _project_body,
        grid=(_PGRID,),
        in_specs=[
            pl.BlockSpec((_HID, _LBLK), lambda i: (0, i)),
            pl.BlockSpec((_HID, _HID), lambda i: (0, 0)),
            pl.BlockSpec((_NCP, _HID), lambda i: (0, 0)),
        ],
        out_specs=pl.BlockSpec((_NCP, _LBLK), lambda i: (0, i)),
        out_shape=jax.ShapeDtypeStruct((_NCP, _VOCABP), jnp.float32),
    )(tabt, w1, w2p)


_VPW = _VOCABP // _NW


def _repack_body(dt_hbm, p_hbm, row_v, out_v):
    c = lax.axis_index("c")
    s = lax.axis_index("s")
    w = s * 2 + c
    base = w * _VPW
    for comp in range(_NCP):
        pltpu.sync_copy(dt_hbm.at[comp, pl.ds(base, _VPW)], row_v.at[comp])
    lane = lax.iota(jnp.int32, 16)

    def body(g, carry):
        rows = g * 16 + lane
        for comp in range(_NCP):
            x = row_v[comp, pl.ds(g * 16, 16)]
            plsc.store_scatter(out_v, [rows, jnp.full((16,), comp, jnp.int32)], x)
        return carry

    lax.fori_loop(0, _VPW // 16, body, 0)
    pltpu.sync_copy(out_v, p_hbm.at[pl.ds(base, _VPW)])


@functools.partial(
    pl.kernel,
    mesh=plsc.VectorSubcoreMesh(core_axis_name="c", subcore_axis_name="s"),
    out_type=jax.ShapeDtypeStruct((_VOCABP, _NCP), jnp.float32),
    compiler_params=pltpu.CompilerParams(
        use_tc_tiling_on_sc=False, needs_layout_passes=False
    ),
    scratch_types=[
        pltpu.VMEM((_NCP, _VPW), jnp.float32),
        pltpu.VMEM((_VPW, _NCP), jnp.float32),
    ],
)
def _repack(dt_hbm, p_hbm, row_v, out_v):
    _repack_body(dt_hbm, p_hbm, row_v, out_v)


def _sc_pool_body(p_hbm, text_hbm, out_hbm, idx_v, buf_v, out_v, sem):
    c = lax.axis_index("c")
    s = lax.axis_index("s")
    w = s * 2 + c

    pltpu.sync_copy(text_hbm.at[pl.ds(w * _CPW, _CPW)], idx_v)

    def issue(j, t):
        return pltpu.async_copy(p_hbm.at[idx_v.at[j]], buf_v.at[t], sem.at[t])

    def wait(j, t):
        pltpu.make_async_copy(p_hbm.at[idx_v.at[j]], buf_v.at[t], sem.at[t]).wait()

    def accumulate(t):
        accs = [buf_v[t, r] for r in range(8)]
        for r in range(8, _CHUNK):
            accs[r % 8] = accs[r % 8] + buf_v[t, r]
        return ((accs[0] + accs[1]) + (accs[2] + accs[3])) + (
            (accs[4] + accs[5]) + (accs[6] + accs[7])
        )

    for t in range(_NBUF - 1):
        issue(jnp.int32(t), t)

    def body(i, carry):
        j0 = i * _NBUF
        for t in range(_NBUF):
            j = j0 + t
            jn = j + (_NBUF - 1)

            @pl.when(jn < _CPW)
            def _():
                issue(jn, (t + _NBUF - 1) % _NBUF)

            wait(j, t)
            half = accumulate(t)
            if t % 2 == 0:
                first = half
            else:
                out_v[(j - 1) // 2] = first + half
        return carry

    lax.fori_loop(0, _CPW // _NBUF, body, 0)
    pltpu.sync_copy(out_v, out_hbm.at[pl.ds(w * _SPW, _SPW)])


@functools.partial(
    pl.kernel,
    mesh=plsc.VectorSubcoreMesh(core_axis_name="c", subcore_axis_name="s"),
    out_type=jax.ShapeDtypeStruct((_B, _NCP), jnp.float32),
    compiler_params=pltpu.CompilerParams(use_tc_tiling_on_sc=False),
    scratch_types=[
        pltpu.VMEM((_CPW, _CHUNK), jnp.int32),
        pltpu.VMEM((_NBUF, _CHUNK, _NCP), jnp.float32),
        pltpu.VMEM((_SPW, _NCP), jnp.float32),
        pltpu.SemaphoreType.DMA((_NBUF,)),
    ],
)
def _sc_pool(p_hbm, text_hbm, out_hbm, idx_v, buf_v, out_v, sem):
    _sc_pool_body(p_hbm, text_hbm, out_hbm, idx_v, buf_v, out_v, sem)


def _finalize_body(zsum_ref, b1_ref, w2p_ref, b2p_ref, out_ref):
    bias = (
        jnp.dot(b1_ref[...], w2p_ref[...].T, preferred_element_type=jnp.float32)
        + b2p_ref[...]
    )
    z = zsum_ref[...] * (1.0 / _L) + bias
    cols = lax.broadcasted_iota(jnp.int32, (_B, _NCP), 1)
    z = jnp.where(cols < _NC, z, -1e30)
    m = jnp.max(z, axis=1, keepdims=True)
    e = jnp.exp(z - m)
    lse = jnp.log(jnp.sum(e, axis=1, keepdims=True))
    out = z - m - lse
    out_ref[...] = out[:, :_NC]


def _finalize(zsum, b1r, w2p, b2p):
    return pl.pallas_call(
        _finalize_body,
        out_shape=jax.ShapeDtypeStruct((_B, _NC), jnp.float32),
    )(zsum, b1r, w2p, b2p)


def kernel(text, text_lengths, table, W1, b1, W2, b2):
    del text_lengths
    w2p = jnp.zeros((_NCP, _HID), jnp.float32).at[:_NC].set(W2)
    b2p = jnp.zeros((1, _NCP), jnp.float32).at[0, :_NC].set(b2)
    dt = _project(table.T, W1, w2p)
    p_lin = _repack(dt)
    zsum = _sc_pool(p_lin, text.reshape(_B * _L // _CHUNK, _CHUNK))
    return _finalize(zsum, b1.reshape(1, _HID), w2p, b2p)


# speedup vs baseline: 1.9108x; 1.0974x over previous
import functools

import jax
import jax.numpy as jnp
from jax import lax
from jax.experimental import pallas as pl
from jax.experimental.pallas import tpu as pltpu
from jax.experimental.pallas import tpu_sc as plsc

_VOCAB = 100000
_VOCABP = 100352
_HID = 64
_NC = 10
_NCP = 16
_B = 4096
_L = 200
_CHUNK = 100

_NW = 32
_SPW = _B // _NW
_CPW = 2 * _SPW
_NBUF = 8

_PACK = _VOCABP // 8
_PGRID = 8
_LBLK = _VOCABP // _PGRID
_PBLK = _LBLK // 8


def _project_body(tabt_ref, w1_ref, w2p_ref, p_ref):
    mt = lax.dot_general(
        w1_ref[...], w2p_ref[...], (((0,), (1,)), ((), ())),
        preferred_element_type=jnp.float32,
    )
    dt = lax.dot_general(
        mt, tabt_ref[...], (((0,), (0,)), ((), ())),
        preferred_element_type=jnp.float32,
    )
    lanes = lax.broadcasted_iota(jnp.int32, (_NCP, _LBLK), 1)
    p_ref[...] = jnp.where((lanes == 0) & (pl.program_id(0) == 0), 0.0, dt)


def _project(tabt, w1, w2p):
    return pl.pallas_call(
        _project_body,
        grid=(_PGRID,),
        in_specs=[
            pl.BlockSpec((_HID, _LBLK), lambda i: (0, i)),
            pl.BlockSpec((_HID, _HID), lambda i: (0, 0)),
            pl.BlockSpec((_NCP, _HID), lambda i: (0, 0)),
        ],
        out_specs=pl.BlockSpec((_NCP, _LBLK), lambda i: (0, i)),
        out_shape=jax.ShapeDtypeStruct((_NCP, _VOCABP), jnp.float32),
    )(tabt, w1, w2p)


_VPW = _VOCABP // _NW


def _repack_body(dt_hbm, p_hbm, row_v, out_v, sem):
    c = lax.axis_index("c")
    s = lax.axis_index("s")
    w = s * 2 + c
    base = w * _VPW
    cps = [
        pltpu.async_copy(dt_hbm.at[comp, pl.ds(base, _VPW)], row_v.at[comp], sem)
        for comp in range(_NCP)
    ]
    for cp in cps:
        cp.wait()
    lane = lax.iota(jnp.int32, 16)

    def body(g, carry):
        rows = g * 16 + lane
        for comp in range(_NCP):
            x = row_v[comp, pl.ds(g * 16, 16)]
            plsc.store_scatter(out_v, [rows, jnp.full((16,), comp, jnp.int32)], x)
        return carry

    lax.fori_loop(0, _VPW // 16, body, 0)
    pltpu.sync_copy(out_v, p_hbm.at[pl.ds(base, _VPW)])


@functools.partial(
    pl.kernel,
    mesh=plsc.VectorSubcoreMesh(core_axis_name="c", subcore_axis_name="s"),
    out_type=jax.ShapeDtypeStruct((_VOCABP, _NCP), jnp.float32),
    compiler_params=pltpu.CompilerParams(
        use_tc_tiling_on_sc=False, needs_layout_passes=False
    ),
    scratch_types=[
        pltpu.VMEM((_NCP, _VPW), jnp.float32),
        pltpu.VMEM((_VPW, _NCP), jnp.float32),
        pltpu.SemaphoreType.DMA,
    ],
)
def _repack(dt_hbm, p_hbm, row_v, out_v, sem):
    _repack_body(dt_hbm, p_hbm, row_v, out_v, sem)


def _sc_pool_body(p_hbm, text_hbm, out_hbm, idx_v, buf_v, out_v, sem):
    c = lax.axis_index("c")
    s = lax.axis_index("s")
    w = s * 2 + c

    pltpu.sync_copy(text_hbm.at[pl.ds(w * _CPW, _CPW)], idx_v)

    def issue(j, t):
        return pltpu.async_copy(p_hbm.at[idx_v.at[j]], buf_v.at[t], sem.at[t])

    def wait(j, t):
        pltpu.make_async_copy(p_hbm.at[idx_v.at[j]], buf_v.at[t], sem.at[t]).wait()

    def accumulate(t):
        accs = [buf_v[t, r] for r in range(8)]
        for r in range(8, _CHUNK):
            accs[r % 8] = accs[r % 8] + buf_v[t, r]
        return ((accs[0] + accs[1]) + (accs[2] + accs[3])) + (
            (accs[4] + accs[5]) + (accs[6] + accs[7])
        )

    for t in range(_NBUF - 1):
        issue(jnp.int32(t), t)

    def body(i, carry):
        j0 = i * _NBUF
        for t in range(_NBUF):
            j = j0 + t
            jn = j + (_NBUF - 1)

            @pl.when(jn < _CPW)
            def _():
                issue(jn, (t + _NBUF - 1) % _NBUF)

            wait(j, t)
            half = accumulate(t)
            if t % 2 == 0:
                first = half
            else:
                out_v[(j - 1) // 2] = first + half
        return carry

    lax.fori_loop(0, _CPW // _NBUF, body, 0)
    pltpu.sync_copy(out_v, out_hbm.at[pl.ds(w * _SPW, _SPW)])


@functools.partial(
    pl.kernel,
    mesh=plsc.VectorSubcoreMesh(core_axis_name="c", subcore_axis_name="s"),
    out_type=jax.ShapeDtypeStruct((_B, _NCP), jnp.float32),
    compiler_params=pltpu.CompilerParams(use_tc_tiling_on_sc=False),
    scratch_types=[
        pltpu.VMEM((_CPW, _CHUNK), jnp.int32),
        pltpu.VMEM((_NBUF, _CHUNK, _NCP), jnp.float32),
        pltpu.VMEM((_SPW, _NCP), jnp.float32),
        pltpu.SemaphoreType.DMA((_NBUF,)),
    ],
)
def _sc_pool(p_hbm, text_hbm, out_hbm, idx_v, buf_v, out_v, sem):
    _sc_pool_body(p_hbm, text_hbm, out_hbm, idx_v, buf_v, out_v, sem)


def _finalize_body(zsum_ref, b1_ref, w2p_ref, b2p_ref, out_ref):
    bias = (
        jnp.dot(b1_ref[...], w2p_ref[...].T, preferred_element_type=jnp.float32)
        + b2p_ref[...]
    )
    z = zsum_ref[...] * (1.0 / _L) + bias
    cols = lax.broadcasted_iota(jnp.int32, (_B, _NCP), 1)
    z = jnp.where(cols < _NC, z, -1e30)
    m = jnp.max(z, axis=1, keepdims=True)
    e = jnp.exp(z - m)
    lse = jnp.log(jnp.sum(e, axis=1, keepdims=True))
    out = z - m - lse
    out_ref[...] = out[:, :_NC]


def _finalize(zsum, b1r, w2p, b2p):
    return pl.pallas_call(
        _finalize_body,
        out_shape=jax.ShapeDtypeStruct((_B, _NC), jnp.float32),
    )(zsum, b1r, w2p, b2p)


def kernel(text, text_lengths, table, W1, b1, W2, b2):
    del text_lengths
    w2p = jnp.zeros((_NCP, _HID), jnp.float32).at[:_NC].set(W2)
    b2p = jnp.zeros((1, _NCP), jnp.float32).at[0, :_NC].set(b2)
    dt = _project(table.T, W1, w2p)
    p_lin = _repack(dt)
    zsum = _sc_pool(p_lin, text.reshape(_B * _L // _CHUNK, _CHUNK))
    return _finalize(zsum, b1.reshape(1, _HID), w2p, b2p)
